# baseline (speedup 1.0000x reference)
import jax
import jax.numpy as jnp
from jax.experimental import pallas as pl

N = 1024
T = 50
IN_DIM = 5
H = 64
GAT_HID = 16

_F32 = jnp.float32


def _leaky(x):
    return jnp.where(x >= 0, x, 0.2 * x)


def _sigmoid(x):
    return 0.5 * jnp.tanh(0.5 * x) + 0.5


def _fused_kernel(xseq_ref, wih_ref, whh_ref, bcol_ref,
                  w1_ref, as1_ref, ad1_ref, b1_ref,
                  w2_ref, as2_ref, ad2_ref, b2_ref,
                  fcw_ref, fcb_ref, out_ref):
    wih = wih_ref[...]
    whh = whh_ref[...]
    bcol = bcol_ref[...]

    def step(xt, hT, cT):
        gx = jax.lax.dot_general(wih, xt, (((1,), (0,)), ((), ())),
                                 preferred_element_type=_F32)
        gh = jax.lax.dot_general(whh, hT, (((1,), (0,)), ((), ())),
                                 preferred_element_type=_F32)
        g = gx + gh + bcol
        i = _sigmoid(g[0:H])
        f = _sigmoid(g[H:2 * H])
        gg = jnp.tanh(g[2 * H:3 * H])
        o = _sigmoid(g[3 * H:4 * H])
        c2 = f * cT + i * gg
        h2 = o * jnp.tanh(c2)
        return h2, c2

    def step2(t2, carry):
        hT, cT = carry
        xpair = xseq_ref[pl.ds(2 * t2, 2)]
        hT, cT = step(xpair[0], hT, cT)
        hT, cT = step(xpair[1], hT, cT)
        return (hT, cT)

    hT0 = jnp.zeros((H, N), _F32)
    cT0 = jnp.zeros((H, N), _F32)
    hT, _ = jax.lax.fori_loop(0, T // 2, step2, (hT0, cT0))

    ones_col = jnp.ones((N, 1), _F32)

    def gat_dense(xp, xpT, att_s, att_d, bias_row, c):
        a_s_col = jax.lax.dot_general(xp, att_s, (((1,), (1,)), ((), ())),
                                      preferred_element_type=_F32)
        a_d_row = jax.lax.dot_general(att_d, xpT, (((1,), (0,)), ((), ())),
                                      preferred_element_type=_F32)
        m_row = _leaky(jnp.max(a_s_col) + a_d_row)
        e = jnp.exp(_leaky(a_s_col + a_d_row) - m_row)
        xp_ext = jnp.concatenate([xp, ones_col], axis=1)
        out_ext = jax.lax.dot_general(e, xp_ext, (((0,), (0,)), ((), ())),
                                      preferred_element_type=_F32)
        out = out_ext[:, :c]
        denom = out_ext[:, c:c + 1]
        return out / (denom + 1e-16) + bias_row

    w1 = w1_ref[...]
    xp1 = jax.lax.dot_general(hT, w1, (((0,), (0,)), ((), ())),
                              preferred_element_type=_F32)
    xp1T = jax.lax.dot_general(w1, hT, (((0,), (0,)), ((), ())),
                               preferred_element_type=_F32)
    h1 = jax.nn.relu(gat_dense(xp1, xp1T, as1_ref[...], ad1_ref[...],
                               b1_ref[...], GAT_HID))

    w2 = w2_ref[...]
    xp2 = jax.lax.dot_general(h1, w2, (((1,), (0,)), ((), ())),
                              preferred_element_type=_F32)
    xp2T = jax.lax.dot_general(w2, h1, (((0,), (1,)), ((), ())),
                               preferred_element_type=_F32)
    out_g = gat_dense(xp2, xp2T, as2_ref[...], ad2_ref[...],
                      b2_ref[...], H)

    out_ext = jnp.concatenate([out_g, ones_col], axis=1)
    fcw_ext = jnp.concatenate([fcw_ref[...], fcb_ref[...]], axis=1)
    pred = _leaky(jax.lax.dot_general(out_ext, fcw_ext,
                                      (((1,), (1,)), ((), ())),
                                      preferred_element_type=_F32))
    out_ref[...] = pred


def kernel(inputs, relation, rel_mask, rel_w, rel_b, W_ih, W_hh, b_ih, b_hh,
           W1, att_s1, att_d1, b1, W2, att_s2, att_d2, b2, fc_w, fc_b):
    xseq = jnp.transpose(inputs, (1, 2, 0))
    bcol = (b_ih + b_hh).reshape(4 * H, 1)
    pred = pl.pallas_call(
        _fused_kernel,
        out_shape=jax.ShapeDtypeStruct((N, 1), _F32),
    )(xseq, W_ih, W_hh, bcol,
      W1, att_s1, att_d1, b1.reshape(1, GAT_HID),
      W2, att_s2, att_d2, b2.reshape(1, H),
      fc_w, fc_b.reshape(1, 1))
    return (pred, rel_w[0, :3])


# speedup vs baseline: 1.2551x; 1.2551x over previous
import jax
import jax.numpy as jnp
from jax.experimental import pallas as pl

N = 1024
T = 50
IN_DIM = 5
H = 64
GAT_HID = 16

_F32 = jnp.float32


def _leaky(x):
    return jnp.where(x >= 0, x, 0.2 * x)


def _sigmoid(x):
    return 0.5 * jnp.tanh(0.5 * x) + 0.5


_UNROLL = 5


def _fused_kernel(xseq_ref, wcat_ref, bcol_ref,
                  w1_ref, as1_ref, ad1_ref, b1_ref,
                  w2_ref, as2_ref, ad2_ref, b2_ref,
                  fcw_ref, fcb_ref, out_ref):
    wcat = wcat_ref[...]
    bcol = bcol_ref[...]

    def step(xt, hT, cT):
        cat = jnp.concatenate([xt, hT], axis=0)
        g = jax.lax.dot_general(wcat, cat, (((1,), (0,)), ((), ())),
                                preferred_element_type=_F32) + bcol
        i = _sigmoid(g[0:H])
        f = _sigmoid(g[H:2 * H])
        gg = jnp.tanh(g[2 * H:3 * H])
        o = _sigmoid(g[3 * H:4 * H])
        c2 = f * cT + i * gg
        h2 = o * jnp.tanh(c2)
        return h2, c2

    def stepu(tu, carry):
        hT, cT = carry
        xs = xseq_ref[pl.ds(_UNROLL * tu, _UNROLL)]
        for k in range(_UNROLL):
            hT, cT = step(xs[k], hT, cT)
        return (hT, cT)

    hT0 = jnp.zeros((H, N), _F32)
    cT0 = jnp.zeros((H, N), _F32)
    hT, _ = jax.lax.fori_loop(0, T // _UNROLL, stepu, (hT0, cT0))

    ones_col = jnp.ones((N, 1), _F32)

    def gat_dense(xp, xpT, att_s, att_d, bias_row, c):
        a_s_col = jax.lax.dot_general(xp, att_s, (((1,), (1,)), ((), ())),
                                      preferred_element_type=_F32)
        a_d_row = jax.lax.dot_general(att_d, xpT, (((1,), (0,)), ((), ())),
                                      preferred_element_type=_F32)
        m_row = _leaky(jnp.max(a_s_col) + a_d_row)
        e = jnp.exp(_leaky(a_s_col + a_d_row) - m_row)
        xp_ext = jnp.concatenate([xp, ones_col], axis=1)
        out_ext = jax.lax.dot_general(e, xp_ext, (((0,), (0,)), ((), ())),
                                      preferred_element_type=_F32)
        out = out_ext[:, :c]
        denom = out_ext[:, c:c + 1]
        return out / (denom + 1e-16) + bias_row

    w1 = w1_ref[...]
    xp1 = jax.lax.dot_general(hT, w1, (((0,), (0,)), ((), ())),
                              preferred_element_type=_F32)
    xp1T = jax.lax.dot_general(w1, hT, (((0,), (0,)), ((), ())),
                               preferred_element_type=_F32)
    h1 = jax.nn.relu(gat_dense(xp1, xp1T, as1_ref[...], ad1_ref[...],
                               b1_ref[...], GAT_HID))

    w2 = w2_ref[...]
    xp2 = jax.lax.dot_general(h1, w2, (((1,), (0,)), ((), ())),
                              preferred_element_type=_F32)
    xp2T = jax.lax.dot_general(w2, h1, (((0,), (1,)), ((), ())),
                               preferred_element_type=_F32)
    out_g = gat_dense(xp2, xp2T, as2_ref[...], ad2_ref[...],
                      b2_ref[...], H)

    out_ext = jnp.concatenate([out_g, ones_col], axis=1)
    fcw_ext = jnp.concatenate([fcw_ref[...], fcb_ref[...]], axis=1)
    pred = _leaky(jax.lax.dot_general(out_ext, fcw_ext,
                                      (((1,), (1,)), ((), ())),
                                      preferred_element_type=_F32))
    out_ref[...] = pred


def kernel(inputs, relation, rel_mask, rel_w, rel_b, W_ih, W_hh, b_ih, b_hh,
           W1, att_s1, att_d1, b1, W2, att_s2, att_d2, b2, fc_w, fc_b):
    xseq = jnp.transpose(inputs, (1, 2, 0))
    wcat = jnp.concatenate([W_ih, W_hh], axis=1)
    bcol = (b_ih + b_hh).reshape(4 * H, 1)
    pred = pl.pallas_call(
        _fused_kernel,
        out_shape=jax.ShapeDtypeStruct((N, 1), _F32),
    )(xseq, wcat, bcol,
      W1, att_s1, att_d1, b1.reshape(1, GAT_HID),
      W2, att_s2, att_d2, b2.reshape(1, H),
      fc_w, fc_b.reshape(1, 1))
    return (pred, rel_w[0, :3])


# speedup vs baseline: 1.2714x; 1.0130x over previous
import jax
import jax.numpy as jnp
from jax.experimental import pallas as pl

N = 1024
T = 50
IN_DIM = 5
H = 64
GAT_HID = 16

_F32 = jnp.float32


def _leaky(x):
    return jnp.where(x >= 0, x, 0.2 * x)


def _sigmoid(x):
    return 0.5 * jnp.tanh(0.5 * x) + 0.5


_UNROLL = 10


def _fused_kernel(xseq_ref, wcat_ref, bcol_ref,
                  w1_ref, as1_ref, ad1_ref, b1_ref,
                  w2_ref, as2_ref, ad2_ref, b2_ref,
                  fcw_ref, fcb_ref, out_ref):
    wcat = wcat_ref[...]
    bcol = bcol_ref[...]

    def step(xt, hT, cT):
        cat = jnp.concatenate([xt, hT], axis=0)
        g = jax.lax.dot_general(wcat, cat, (((1,), (0,)), ((), ())),
                                preferred_element_type=_F32) + bcol
        i = _sigmoid(g[0:H])
        f = _sigmoid(g[H:2 * H])
        gg = jnp.tanh(g[2 * H:3 * H])
        o = _sigmoid(g[3 * H:4 * H])
        c2 = f * cT + i * gg
        h2 = o * jnp.tanh(c2)
        return h2, c2

    def stepu(tu, carry):
        hT, cT = carry
        xs = xseq_ref[pl.ds(_UNROLL * tu, _UNROLL)]
        for k in range(_UNROLL):
            hT, cT = step(xs[k], hT, cT)
        return (hT, cT)

    hT0 = jnp.zeros((H, N), _F32)
    cT0 = jnp.zeros((H, N), _F32)
    hT, _ = jax.lax.fori_loop(0, T // _UNROLL, stepu, (hT0, cT0))

    ones_col = jnp.ones((N, 1), _F32)

    def gat_dense(xp, xpT, att_s, att_d, bias_row, c):
        a_s_col = jax.lax.dot_general(xp, att_s, (((1,), (1,)), ((), ())),
                                      preferred_element_type=_F32)
        a_d_row = jax.lax.dot_general(att_d, xpT, (((1,), (0,)), ((), ())),
                                      preferred_element_type=_F32)
        m_row = _leaky(jnp.max(a_s_col) + a_d_row)
        e = jnp.exp(_leaky(a_s_col + a_d_row) - m_row)
        xp_ext = jnp.concatenate([xp, ones_col], axis=1)
        out_ext = jax.lax.dot_general(e, xp_ext, (((0,), (0,)), ((), ())),
                                      preferred_element_type=_F32)
        out = out_ext[:, :c]
        denom = out_ext[:, c:c + 1]
        return out / (denom + 1e-16) + bias_row

    w1 = w1_ref[...]
    xp1 = jax.lax.dot_general(hT, w1, (((0,), (0,)), ((), ())),
                              preferred_element_type=_F32)
    xp1T = jax.lax.dot_general(w1, hT, (((0,), (0,)), ((), ())),
                               preferred_element_type=_F32)
    h1 = jax.nn.relu(gat_dense(xp1, xp1T, as1_ref[...], ad1_ref[...],
                               b1_ref[...], GAT_HID))

    w2 = w2_ref[...]
    xp2 = jax.lax.dot_general(h1, w2, (((1,), (0,)), ((), ())),
                              preferred_element_type=_F32)
    xp2T = jax.lax.dot_general(w2, h1, (((0,), (1,)), ((), ())),
                               preferred_element_type=_F32)
    out_g = gat_dense(xp2, xp2T, as2_ref[...], ad2_ref[...],
                      b2_ref[...], H)

    out_ext = jnp.concatenate([out_g, ones_col], axis=1)
    fcw_ext = jnp.concatenate([fcw_ref[...], fcb_ref[...]], axis=1)
    pred = _leaky(jax.lax.dot_general(out_ext, fcw_ext,
                                      (((1,), (1,)), ((), ())),
                                      preferred_element_type=_F32))
    out_ref[...] = pred


def kernel(inputs, relation, rel_mask, rel_w, rel_b, W_ih, W_hh, b_ih, b_hh,
           W1, att_s1, att_d1, b1, W2, att_s2, att_d2, b2, fc_w, fc_b):
    xseq = jnp.transpose(inputs, (1, 2, 0))
    wcat = jnp.concatenate([W_ih, W_hh], axis=1)
    bcol = (b_ih + b_hh).reshape(4 * H, 1)
    pred = pl.pallas_call(
        _fused_kernel,
        out_shape=jax.ShapeDtypeStruct((N, 1), _F32),
    )(xseq, wcat, bcol,
      W1, att_s1, att_d1, b1.reshape(1, GAT_HID),
      W2, att_s2, att_d2, b2.reshape(1, H),
      fc_w, fc_b.reshape(1, 1))
    return (pred, rel_w[0, :3])
